# baseline (speedup 1.0000x reference)
import functools

import jax
import jax.numpy as jnp
import numpy as np
from jax import lax
from jax.experimental import pallas as pl
from jax.experimental.pallas import tpu as pltpu
from jax.experimental.pallas import tpu_sc as plsc

B, N, K, D, H = 2, 8192, 16, 128, 8
HD = D // H
TOK = B * N
NC, NS = 2, 16
NW = NC * NS
TPW = TOK // NW
CT = 8
NCHUNK = TPW // CT
GI = CT * K
DKV = 2 * D


def _tree_reduce(xs, op):
    xs = list(xs)
    while len(xs) > 1:
        xs = [op(xs[i], xs[i + 1]) for i in range(0, len(xs) - 1, 2)] + (
            [xs[-1]] if len(xs) % 2 else [])
    return xs[0]


def _mirror(v):
    return lax.rev(v, (0,))


def _sc_attn_body(xq_hbm, xkv_hbm, idx_hbm, out_hbm,
                  idx3_v, idx_v, xkv_v, xq_v, out_v, isem, gsem, osem):
    wid = lax.axis_index("s") * NC + lax.axis_index("c")

    def idx_copy(c, p):
        base = wid * TPW + c * CT
        return pltpu.make_async_copy(idx_hbm.at[pl.ds(base, CT)],
                                     idx3_v.at[p], isem.at[p])

    def gather_copies(c, p):
        base = wid * TPW + c * CT
        return (
            pltpu.make_async_copy(xkv_hbm.at[idx_v.at[p]], xkv_v.at[p],
                                  gsem.at[p]),
            pltpu.make_async_copy(xq_hbm.at[pl.ds(base, CT)], xq_v.at[p],
                                  gsem.at[p]),
        )

    def out_copy(c, p):
        base = wid * TPW + c * CT
        return pltpu.make_async_copy(out_v.at[p], out_hbm.at[pl.ds(base, CT)],
                                     osem.at[p])

    woff = jnp.where(wid >= NS, jnp.int32(N), jnp.int32(0))

    def add_offset(p):
        for t in range(CT):
            idx_v[p, pl.ds(16 * t, 16)] = idx3_v[p, t, :] + woff

    def compute(p):
        def token_body(t, _):
            q = [xq_v[p, t, pl.ds(16 * j, 16)] for j in range(8)]
            logits = []
            for k in range(K):
                row = t * K + k
                acc = _tree_reduce(
                    [q[j] * xkv_v[p, row, pl.ds(16 * j, 16)]
                     for j in range(8)], lax.add)
                logits.append(acc + _mirror(acc))
            mx = _tree_reduce(logits, jnp.maximum)
            es = [jnp.exp(l - mx) for l in logits]
            ssum = _tree_reduce(es, lax.add)
            rinv = 1.0 / ssum
            acc_o = [None] * 8
            for k in range(K):
                row = t * K + k
                a = es[k] * rinv
                for j in range(8):
                    term = a * xkv_v[p, row, pl.ds(D + 16 * j, 16)]
                    acc_o[j] = term if k == 0 else acc_o[j] + term
            for j in range(8):
                out_v[p, t, pl.ds(16 * j, 16)] = acc_o[j]
            return 0

        lax.fori_loop(0, CT, token_body, 0)

    idx_copy(0, 0).start()
    idx_copy(0, 0).wait()
    add_offset(0)
    for cp in gather_copies(0, 0):
        cp.start()
    idx_copy(1, 1).start()

    def pair_body(i, _):
        c0 = 2 * i
        last = NCHUNK // 2 - 1

        idx_copy(c0 + 1, 1).wait()
        add_offset(1)
        for cp in gather_copies(c0 + 1, 1):
            cp.start()
        for cp in gather_copies(c0, 0):
            cp.wait()

        @pl.when(i < last)
        def _():
            idx_copy(c0 + 2, 0).start()

        @pl.when(i > 0)
        def _():
            out_copy(c0 - 2, 0).wait()

        compute(0)
        out_copy(c0, 0).start()

        @pl.when(i < last)
        def _():
            idx_copy(c0 + 2, 0).wait()
            add_offset(0)
            for cp in gather_copies(c0 + 2, 0):
                cp.start()

        for cp in gather_copies(c0 + 1, 1):
            cp.wait()

        @pl.when(i < last)
        def _():
            idx_copy(c0 + 3, 1).start()

        @pl.when(i > 0)
        def _():
            out_copy(c0 - 1, 1).wait()

        compute(1)
        out_copy(c0 + 1, 1).start()
        return 0

    lax.fori_loop(0, NCHUNK // 2, pair_body, 0)
    out_copy(NCHUNK - 2, 0).wait()
    out_copy(NCHUNK - 1, 1).wait()


_sc_attn = functools.partial(
    pl.kernel,
    out_type=jax.ShapeDtypeStruct((TOK, D), jnp.float32),
    mesh=plsc.VectorSubcoreMesh(
        core_axis_name="c", subcore_axis_name="s", num_cores=NC,
        num_subcores=NS),
    scratch_types=[
        pltpu.VMEM((2, CT, K), jnp.int32),
        pltpu.VMEM((2, GI), jnp.int32),
        pltpu.VMEM((2, GI, DKV), jnp.float32),
        pltpu.VMEM((2, CT, D), jnp.float32),
        pltpu.VMEM((2, CT, D), jnp.float32),
        pltpu.SemaphoreType.DMA((2,)),
        pltpu.SemaphoreType.DMA((2,)),
        pltpu.SemaphoreType.DMA((2,)),
    ],
)(_sc_attn_body)


def _qkv_body(x_ref, w_ref, xq_ref, xkv_ref):
    y = lax.dot_general(x_ref[...], w_ref[...], (((1,), (0,)), ((), ())),
                        preferred_element_type=jnp.float32)
    xq_ref[...] = y[:, :D]
    xkv_ref[...] = y[:, D:]


def _epi_body(x_ref, a_ref, xv_ref, wo_ref, o_ref):
    d = a_ref[...] - xv_ref[...]
    o_ref[...] = x_ref[...] + lax.dot_general(
        d, wo_ref[...], (((1,), (0,)), ((), ())),
        preferred_element_type=jnp.float32)


_BLK = 2048
_row_spec = pl.BlockSpec((_BLK, D), lambda i: (i, 0))
_full_spec = pl.BlockSpec((D, D), lambda i: (0, 0))

_qkv_call = pl.pallas_call(
    _qkv_body,
    grid=(TOK // _BLK,),
    in_specs=[_row_spec, pl.BlockSpec((D, 3 * D), lambda i: (0, 0))],
    out_specs=[_row_spec, pl.BlockSpec((_BLK, DKV), lambda i: (i, 0))],
    out_shape=[jax.ShapeDtypeStruct((TOK, D), jnp.float32),
               jax.ShapeDtypeStruct((TOK, DKV), jnp.float32)],
)

_epi_call = pl.pallas_call(
    _epi_body,
    grid=(TOK // _BLK,),
    in_specs=[_row_spec, _row_spec,
              pl.BlockSpec((_BLK, D), lambda i: (i, 1)), _full_spec],
    out_specs=_row_spec,
    out_shape=jax.ShapeDtypeStruct((TOK, D), jnp.float32),
)

_l = np.arange(D)
_j, _r = _l // 16, _l % 16
_h = np.where(_r < 8, _r, 15 - _r)
_p = (_r >= 8).astype(np.int64)
_PERM = np.asarray(_h * HD + 2 * _j + _p, dtype=np.int64)


def kernel(x, idx, in_proj_weight, out_proj_weight):
    x2 = x.reshape(TOK, D)
    scale = 1.0 / np.sqrt(HD)
    wq = in_proj_weight[:D][_PERM, :] * scale
    wk = in_proj_weight[D:2 * D][_PERM, :]
    wv = in_proj_weight[2 * D:][_PERM, :]
    wcat = jnp.concatenate([wq, wk, wv], axis=0).T
    wo_t = out_proj_weight[:, _PERM].T

    xq, xkv = _qkv_call(x2, wcat)

    a = _sc_attn(xq, xkv, idx.reshape(TOK, K))
    out = _epi_call(x2, a, xkv, wo_t)
    return out.reshape(B, N, D)


# speedup vs baseline: 1.0108x; 1.0108x over previous
import functools

import jax
import jax.numpy as jnp
import numpy as np
from jax import lax
from jax.experimental import pallas as pl
from jax.experimental.pallas import tpu as pltpu
from jax.experimental.pallas import tpu_sc as plsc

B, N, K, D, H = 2, 8192, 16, 128, 8
HD = D // H
TOK = B * N
NC, NS = 2, 16
NW = NC * NS
TPW = TOK // NW
CT = 8
NCHUNK = TPW // CT
GI = CT * K
DKV = 2 * D


def _tree_reduce(xs, op):
    xs = list(xs)
    while len(xs) > 1:
        xs = [op(xs[i], xs[i + 1]) for i in range(0, len(xs) - 1, 2)] + (
            [xs[-1]] if len(xs) % 2 else [])
    return xs[0]


def _mirror(v):
    return lax.rev(v, (0,))


def _sc_attn_body(xq_hbm, xkv_hbm, idx_hbm, out_hbm,
                  idx3_v, idx_v, xkv_v, xq_v, out_v, isem, gsem, osem):
    wid = lax.axis_index("s") * NC + lax.axis_index("c")

    def idx_copy(c, p):
        base = wid * TPW + c * CT
        return pltpu.make_async_copy(idx_hbm.at[pl.ds(base, CT)],
                                     idx3_v.at[p], isem.at[p])

    def gather_copies(c, p):
        base = wid * TPW + c * CT
        return (
            pltpu.make_async_copy(xkv_hbm.at[idx_v.at[p]], xkv_v.at[p],
                                  gsem.at[p]),
            pltpu.make_async_copy(xq_hbm.at[pl.ds(base, CT)], xq_v.at[p],
                                  gsem.at[p]),
        )

    def out_copy(c, p):
        base = wid * TPW + c * CT
        return pltpu.make_async_copy(out_v.at[p], out_hbm.at[pl.ds(base, CT)],
                                     osem.at[p])

    woff = jnp.where(wid >= NS, jnp.int32(N), jnp.int32(0))

    def add_offset(p):
        for t in range(CT):
            idx_v[p, pl.ds(16 * t, 16)] = idx3_v[p, t, :] + woff

    def compute(p):
        def token_body(t, _):
            q = [xq_v[p, t, pl.ds(16 * j, 16)] for j in range(8)]
            logits = []
            for k in range(K):
                row = t * K + k
                acc = _tree_reduce(
                    [q[j] * xkv_v[p, row, pl.ds(16 * j, 16)]
                     for j in range(8)], lax.add)
                logits.append(acc + _mirror(acc))
            mx = _tree_reduce(logits, jnp.maximum)
            es = [jnp.exp(l - mx) for l in logits]
            ssum = _tree_reduce(es, lax.add)
            rinv = 1.0 / ssum
            acc_o = [None] * 8
            for k in range(K):
                row = t * K + k
                for j in range(8):
                    term = es[k] * xkv_v[p, row, pl.ds(D + 16 * j, 16)]
                    acc_o[j] = term if k == 0 else acc_o[j] + term
            for j in range(8):
                out_v[p, t, pl.ds(16 * j, 16)] = acc_o[j] * rinv
            return 0

        lax.fori_loop(0, CT, token_body, 0)

    idx_copy(0, 0).start()
    idx_copy(0, 0).wait()
    add_offset(0)
    for cp in gather_copies(0, 0):
        cp.start()
    idx_copy(1, 1).start()

    def pair_body(i, _):
        c0 = 2 * i
        last = NCHUNK // 2 - 1

        idx_copy(c0 + 1, 1).wait()
        add_offset(1)
        for cp in gather_copies(c0 + 1, 1):
            cp.start()
        for cp in gather_copies(c0, 0):
            cp.wait()

        @pl.when(i < last)
        def _():
            idx_copy(c0 + 2, 0).start()

        @pl.when(i > 0)
        def _():
            out_copy(c0 - 2, 0).wait()

        compute(0)
        out_copy(c0, 0).start()

        @pl.when(i < last)
        def _():
            idx_copy(c0 + 2, 0).wait()
            add_offset(0)
            for cp in gather_copies(c0 + 2, 0):
                cp.start()

        for cp in gather_copies(c0 + 1, 1):
            cp.wait()

        @pl.when(i < last)
        def _():
            idx_copy(c0 + 3, 1).start()

        @pl.when(i > 0)
        def _():
            out_copy(c0 - 1, 1).wait()

        compute(1)
        out_copy(c0 + 1, 1).start()
        return 0

    lax.fori_loop(0, NCHUNK // 2, pair_body, 0)
    out_copy(NCHUNK - 2, 0).wait()
    out_copy(NCHUNK - 1, 1).wait()


_sc_attn = functools.partial(
    pl.kernel,
    out_type=jax.ShapeDtypeStruct((TOK, D), jnp.float32),
    mesh=plsc.VectorSubcoreMesh(
        core_axis_name="c", subcore_axis_name="s", num_cores=NC,
        num_subcores=NS),
    scratch_types=[
        pltpu.VMEM((2, CT, K), jnp.int32),
        pltpu.VMEM((2, GI), jnp.int32),
        pltpu.VMEM((2, GI, DKV), jnp.float32),
        pltpu.VMEM((2, CT, D), jnp.float32),
        pltpu.VMEM((2, CT, D), jnp.float32),
        pltpu.SemaphoreType.DMA((2,)),
        pltpu.SemaphoreType.DMA((2,)),
        pltpu.SemaphoreType.DMA((2,)),
    ],
)(_sc_attn_body)


def _qkv_body(x_ref, w_ref, xq_ref, xkv_ref):
    y = lax.dot_general(x_ref[...], w_ref[...], (((1,), (0,)), ((), ())),
                        preferred_element_type=jnp.float32)
    xq_ref[...] = y[:, :D]
    xkv_ref[...] = y[:, D:]


def _epi_body(x_ref, a_ref, xv_ref, wo_ref, o_ref):
    d = a_ref[...] - xv_ref[...]
    o_ref[...] = x_ref[...] + lax.dot_general(
        d, wo_ref[...], (((1,), (0,)), ((), ())),
        preferred_element_type=jnp.float32)


_BLK = 2048
_row_spec = pl.BlockSpec((_BLK, D), lambda i: (i, 0))
_full_spec = pl.BlockSpec((D, D), lambda i: (0, 0))

_qkv_call = pl.pallas_call(
    _qkv_body,
    grid=(TOK // _BLK,),
    in_specs=[_row_spec, pl.BlockSpec((D, 3 * D), lambda i: (0, 0))],
    out_specs=[_row_spec, pl.BlockSpec((_BLK, DKV), lambda i: (i, 0))],
    out_shape=[jax.ShapeDtypeStruct((TOK, D), jnp.float32),
               jax.ShapeDtypeStruct((TOK, DKV), jnp.float32)],
)

_epi_call = pl.pallas_call(
    _epi_body,
    grid=(TOK // _BLK,),
    in_specs=[_row_spec, _row_spec,
              pl.BlockSpec((_BLK, D), lambda i: (i, 1)), _full_spec],
    out_specs=_row_spec,
    out_shape=jax.ShapeDtypeStruct((TOK, D), jnp.float32),
)

_l = np.arange(D)
_j, _r = _l // 16, _l % 16
_h = np.where(_r < 8, _r, 15 - _r)
_p = (_r >= 8).astype(np.int64)
_PERM = np.asarray(_h * HD + 2 * _j + _p, dtype=np.int64)


def kernel(x, idx, in_proj_weight, out_proj_weight):
    x2 = x.reshape(TOK, D)
    scale = 1.0 / np.sqrt(HD)
    wq = in_proj_weight[:D][_PERM, :] * scale
    wk = in_proj_weight[D:2 * D][_PERM, :]
    wv = in_proj_weight[2 * D:][_PERM, :]
    wcat = jnp.concatenate([wq, wk, wv], axis=0).T
    wo_t = out_proj_weight[:, _PERM].T

    xq, xkv = _qkv_call(x2, wcat)

    a = _sc_attn(xq, xkv, idx.reshape(TOK, K))
    out = _epi_call(x2, a, xkv, wo_t)
    return out.reshape(B, N, D)
